# Initial kernel scaffold; baseline (speedup 1.0000x reference)
import jax
import jax.numpy as jnp
from jax.experimental import pallas as pl


def kernel(x, edge_index, batch, Wl1, bl1, Wr1, br1, att1, bias1, Wl2, bl2, Wr2, br2, att2, bias2, Wl3, bl3, Wr3, br3, att3, bias3, W1, b1, W2, b2):
    raise NotImplementedError("write your pallas kernel here")


# speedup vs baseline: 1.0040x; 1.0040x over previous
import functools

import jax
import jax.numpy as jnp
from jax.experimental import pallas as pl
from jax.experimental.pallas import tpu as pltpu


def _mm2_body(h_ref, wl_ref, bl_ref, wr_ref, br_ref, xl_ref, xr_ref):
    h = h_ref[...]
    xl_ref[...] = h @ wl_ref[...] + bl_ref[...]
    xr_ref[...] = h @ wr_ref[...] + br_ref[...]


def _mm2(h, Wl, bl, Wr, br):
    n, dout = h.shape[0], Wl.shape[1]
    return pl.pallas_call(
        _mm2_body,
        out_shape=[
            jax.ShapeDtypeStruct((n, dout), jnp.float32),
            jax.ShapeDtypeStruct((n, dout), jnp.float32),
        ],
    )(h, Wl, bl.reshape(1, -1), Wr, br.reshape(1, -1))


def _mlp_body(p_ref, w1_ref, b1_ref, w2_ref, b2_ref, o_ref):
    h = jnp.maximum(p_ref[...] @ w1_ref[...] + b1_ref[...], 0.0)
    o_ref[...] = h @ w2_ref[...] + b2_ref[...]


def _edge_phase(xl, xr, att, src, dst, n):
    e = jax.nn.leaky_relu(xl[src] + xr[dst], negative_slope=0.2)
    logits = e @ att
    m = jax.ops.segment_max(logits, dst, num_segments=n)
    m = jnp.where(jnp.isfinite(m), m, 0.0)
    ex = jnp.exp(logits - m[dst])
    denom = jax.ops.segment_sum(ex, dst, num_segments=n)
    alpha = ex / jnp.maximum(denom[dst], 1e-16)
    return jax.ops.segment_sum(alpha[:, None] * xl[src], dst, num_segments=n)


def kernel(x, edge_index, batch, Wl1, bl1, Wr1, br1, att1, bias1, Wl2, bl2,
           Wr2, br2, att2, bias2, Wl3, bl3, Wr3, br3, att3, bias3, W1, b1,
           W2, b2):
    n = x.shape[0]
    n_graphs = 256
    loop = jnp.arange(n, dtype=edge_index.dtype)
    src = jnp.concatenate([edge_index[0], loop])
    dst = jnp.concatenate([edge_index[1], loop])

    h = x
    for (Wl, bl, Wr, br, att, bias, act) in (
        (Wl1, bl1, Wr1, br1, att1, bias1, True),
        (Wl2, bl2, Wr2, br2, att2, bias2, True),
        (Wl3, bl3, Wr3, br3, att3, bias3, False),
    ):
        xl, xr = _mm2(h, Wl, bl, Wr, br)
        out = _edge_phase(xl, xr, att, src, dst, n) + bias
        h = jnp.maximum(out, 0.0) if act else out

    sums = jax.ops.segment_sum(h, batch, num_segments=n_graphs)
    counts = jax.ops.segment_sum(jnp.ones((n,), jnp.float32), batch,
                                 num_segments=n_graphs)
    pooled = sums / jnp.maximum(counts[:, None], 1.0)

    return pl.pallas_call(
        _mlp_body,
        out_shape=jax.ShapeDtypeStruct((n_graphs, 1), jnp.float32),
    )(pooled, W1, b1.reshape(1, -1), W2, b2.reshape(1, -1))


# speedup vs baseline: 8.8354x; 8.8006x over previous
import functools

import jax
import jax.numpy as jnp
from jax import lax
from jax.experimental import pallas as pl
from jax.experimental.pallas import tpu as pltpu
from jax.experimental.pallas import tpu_sc as plsc

N = 10000
N_GRAPHS = 256
PADROWS = 112
NROWS = N + PADROWS
ZPER = NROWS // 16
NW = 32
CHUNK = 128
E_TOT = 640000 + N
EPW = 20480
NCHUNK = EPW // CHUNK
E_PAD = NW * EPW

_mesh = plsc.VectorSubcoreMesh(core_axis_name="c", subcore_axis_name="s")

_GDN = lax.GatherDimensionNumbers(
    offset_dims=(), collapsed_slice_dims=(0,), start_index_map=(0,))


def _permute(v, idx):
    return lax.gather(v, idx[:, None], _GDN, slice_sizes=(1,),
                      mode=lax.GatherScatterMode.PROMISE_IN_BOUNDS)


def _vsum16(v, lane):
    for s in (8, 4, 2, 1):
        v = v + _permute(v, lane ^ s)
    return v


def _vmax16(v, lane):
    for s in (8, 4, 2, 1):
        v = jnp.maximum(v, _permute(v, lane ^ s))
    return v


def _tc_prep_body(h_ref, wl_ref, bl_ref, wr_ref, br_ref, xl_ref, xr_ref):
    h = h_ref[...]
    R = xl_ref.shape[1]
    pad = jnp.zeros((PADROWS, R), jnp.float32)
    xl = h @ wl_ref[...] + bl_ref[...]
    xr = h @ wr_ref[...] + br_ref[...]
    xl_ref[...] = jnp.concatenate([xl, pad], axis=0)
    xr_ref[...] = jnp.concatenate([xr, pad], axis=0)


def _tc_prep(h, Wlp, blp, Wrp, brp, R):
    return pl.pallas_call(
        _tc_prep_body,
        out_shape=[
            jax.ShapeDtypeStruct((NROWS, R), jnp.float32),
            jax.ShapeDtypeStruct((NROWS, R), jnp.float32),
        ],
    )(h, Wlp, blp, Wrp, brp)


def _combine(acc_ref, bias_ref, d):
    a = acc_ref[0] + acc_ref[1]
    num = lax.slice(a, (0, 0), (N, d))
    den = lax.slice(a, (0, d), (N, d + 1))
    return num / jnp.maximum(den, 1e-16) + bias_ref[...]


def _tc_comb_prep_body(acc_ref, bias_ref, wl_ref, bl_ref, wr_ref, br_ref,
                       xl_ref, xr_ref, *, d):
    h = jnp.maximum(_combine(acc_ref, bias_ref, d), 0.0)
    _tc_prep_body_inner(h, wl_ref, bl_ref, wr_ref, br_ref, xl_ref, xr_ref)


def _tc_prep_body_inner(h, wl_ref, bl_ref, wr_ref, br_ref, xl_ref, xr_ref):
    R = xl_ref.shape[1]
    pad = jnp.zeros((PADROWS, R), jnp.float32)
    xl = h @ wl_ref[...] + bl_ref[...]
    xr = h @ wr_ref[...] + br_ref[...]
    xl_ref[...] = jnp.concatenate([xl, pad], axis=0)
    xr_ref[...] = jnp.concatenate([xr, pad], axis=0)


def _tc_comb_prep(acc, bias, Wlp, blp, Wrp, brp, d, R):
    return pl.pallas_call(
        functools.partial(_tc_comb_prep_body, d=d),
        out_shape=[
            jax.ShapeDtypeStruct((NROWS, R), jnp.float32),
            jax.ShapeDtypeStruct((NROWS, R), jnp.float32),
        ],
    )(acc, bias, Wlp, blp, Wrp, brp)


def _tc_final_body(acc_ref, bias_ref, batch_ref, w1_ref, b1_ref, w2_ref,
                   b2_ref, o_ref):
    h = _combine(acc_ref, bias_ref, 30)
    seg = lax.broadcasted_iota(jnp.int32, (N, N_GRAPHS), 1)
    p = jnp.where(batch_ref[...] == seg, 1.0, 0.0)
    dn = (((0,), (0,)), ((), ()))
    pooled = lax.dot_general(p, h, dn, preferred_element_type=jnp.float32)
    counts = lax.dot_general(p, jnp.ones((N, 1), jnp.float32), dn,
                             preferred_element_type=jnp.float32)
    pooled = pooled / jnp.maximum(counts, 1.0)
    hid = jnp.maximum(pooled @ w1_ref[...] + b1_ref[...], 0.0)
    o_ref[...] = hid @ w2_ref[...] + b2_ref[...]


def _tc_final(acc, bias, batch_col, W1, b1, W2, b2):
    return pl.pallas_call(
        _tc_final_body,
        out_shape=jax.ShapeDtypeStruct((N_GRAPHS, 1), jnp.float32),
    )(acc, bias, batch_col, W1, b1, W2, b2)


def _make_passA(R):
    KV = R // 16

    @functools.partial(
        pl.kernel,
        mesh=_mesh,
        out_type=[
            jax.ShapeDtypeStruct((E_PAD,), jnp.float32),
            jax.ShapeDtypeStruct((NW, 16), jnp.float32),
        ],
        scratch_types=[
            pltpu.VMEM((CHUNK,), jnp.int32),
            pltpu.VMEM((CHUNK,), jnp.int32),
            pltpu.VMEM((CHUNK, R), jnp.float32),
            pltpu.VMEM((CHUNK, R), jnp.float32),
            pltpu.VMEM((CHUNK,), jnp.float32),
            pltpu.VMEM((16,), jnp.float32),
            pltpu.VMEM((R,), jnp.float32),
            pltpu.SemaphoreType.DMA,
            pltpu.SemaphoreType.DMA,
        ],
    )
    def passA(src_hbm, dst_hbm, xl_hbm, xr_hbm, att_hbm, logits_hbm,
              wmax_hbm, srcv, dstv, rowsA, rowsB, lv, mv, attv, semA, semB):
        wid = lax.axis_index("s") * 2 + lax.axis_index("c")
        pltpu.sync_copy(att_hbm, attv)
        att_regs = [attv[pl.ds(16 * k, 16)] for k in range(KV)]

        lane = lax.broadcasted_iota(jnp.int32, (16,), 0)

        def chunk_body(c, smaxvec):
            base = wid * EPW + c * CHUNK
            pltpu.sync_copy(src_hbm.at[pl.ds(base, CHUNK)], srcv)
            pltpu.sync_copy(dst_hbm.at[pl.ds(base, CHUNK)], dstv)
            cpa = pltpu.async_copy(xl_hbm.at[srcv], rowsA, semA)
            cpb = pltpu.async_copy(xr_hbm.at[dstv], rowsB, semB)
            cpa.wait()
            cpb.wait()

            def group_body(t, smv):
                lgvec = jnp.zeros((16,), jnp.float32)
                for jj in range(16):
                    j = 16 * t + jj
                    acc = jnp.zeros((16,), jnp.float32)
                    for k in range(KV):
                        u = (rowsA[j, pl.ds(16 * k, 16)]
                             + rowsB[j, pl.ds(16 * k, 16)])
                        e = jnp.maximum(u, 0.2 * u)
                        acc = acc + e * att_regs[k]
                    lg = _vsum16(acc, lane)
                    lgvec = jnp.where(lane == jj, lg, lgvec)
                lv[pl.ds(16 * t, 16)] = lgvec
                return jnp.maximum(smv, lgvec)

            smaxvec = lax.fori_loop(0, CHUNK // 16, group_body, smaxvec)
            pltpu.sync_copy(lv, logits_hbm.at[pl.ds(base, CHUNK)])
            return smaxvec

        smaxvec = lax.fori_loop(0, NCHUNK, chunk_body,
                                jnp.full((16,), -1e30, jnp.float32))
        mv[...] = smaxvec
        pltpu.sync_copy(mv, wmax_hbm.at[wid])

    return passA


def _make_passB(R):
    KV = R // 16

    @functools.partial(
        pl.kernel,
        mesh=_mesh,
        out_type=[jax.ShapeDtypeStruct((2 * NROWS, R), jnp.float32)],
        scratch_types=[
            pltpu.VMEM((CHUNK,), jnp.int32),
            pltpu.VMEM((CHUNK,), jnp.int32),
            pltpu.VMEM((CHUNK, R), jnp.float32),
            pltpu.VMEM((CHUNK,), jnp.float32),
            pltpu.VMEM((NW, 16), jnp.float32),
            pltpu.VMEM_SHARED((NROWS, R), jnp.float32),
            pltpu.SemaphoreType.DMA,
        ],
    )
    def passB(src_hbm, dst_hbm, logits_hbm, wmax_hbm, xl_hbm, zeros_hbm,
              acc_hbm, srcv, dstv, rows, lv, mw, acc_sh, semA):
        cid = lax.axis_index("c")
        sid = lax.axis_index("s")
        wid = sid * 2 + cid
        lane = lax.broadcasted_iota(jnp.int32, (16,), 0)
        pltpu.sync_copy(wmax_hbm, mw)
        mvec = mw[0, pl.ds(0, 16)]
        for k in range(1, NW):
            mvec = jnp.maximum(mvec, mw[k, pl.ds(0, 16)])
        mfull = _vmax16(mvec, lane)

        pltpu.sync_copy(zeros_hbm, acc_sh.at[pl.ds(sid * ZPER, ZPER)])
        plsc.subcore_barrier()

        def chunk_body(c, carry):
            base = wid * EPW + c * CHUNK
            pltpu.sync_copy(src_hbm.at[pl.ds(base, CHUNK)], srcv)
            pltpu.sync_copy(dst_hbm.at[pl.ds(base, CHUNK)], dstv)
            pltpu.sync_copy(logits_hbm.at[pl.ds(base, CHUNK)], lv)
            cpa = pltpu.async_copy(xl_hbm.at[srcv], rows, semA)
            for t in range(CHUNK // 16):
                lv[pl.ds(16 * t, 16)] = jnp.exp(lv[pl.ds(16 * t, 16)] - mfull)
            cpa.wait()

            def group_body(t, _):
                exvec = lv[pl.ds(16 * t, 16)]
                for jj in range(16):
                    j = 16 * t + jj
                    s = _permute(exvec, jnp.full((16,), jj, jnp.int32))
                    for k in range(KV):
                        rows[j, pl.ds(16 * k, 16)] = (
                            rows[j, pl.ds(16 * k, 16)] * s)
                return 0

            lax.fori_loop(0, CHUNK // 16, group_body, 0)
            pltpu.sync_copy(rows, acc_sh.at[dstv], add=True)
            return carry

        lax.fori_loop(0, NCHUNK, chunk_body, 0)
        plsc.subcore_barrier()
        pltpu.sync_copy(
            acc_sh.at[pl.ds(sid * ZPER, ZPER)],
            acc_hbm.at[pl.ds(cid * NROWS + sid * ZPER, ZPER)],
        )

    return passB


def _pad_w(W, b, R, ones_col):
    din, d = W.shape
    Wp = jnp.zeros((din, R), jnp.float32).at[:, :d].set(W)
    bp = jnp.zeros((R,), jnp.float32).at[:d].set(b)
    if ones_col:
        bp = bp.at[d].set(1.0)
    return Wp, bp.reshape(1, R)


def kernel(x, edge_index, batch, Wl1, bl1, Wr1, br1, att1, bias1, Wl2, bl2,
           Wr2, br2, att2, bias2, Wl3, bl3, Wr3, br3, att3, bias3, W1, b1,
           W2, b2):
    loop = jnp.arange(N, dtype=edge_index.dtype)
    fill = jnp.full((E_PAD - E_TOT,), N, jnp.int32)
    src = jnp.concatenate([edge_index[0], loop, fill])
    dst = jnp.concatenate([edge_index[1], loop, fill])

    Rs = (128, 128, 128)
    ds = (100, 60, 30)
    layers = (
        (Wl1, bl1, Wr1, br1, att1, bias1),
        (Wl2, bl2, Wr2, br2, att2, bias2),
        (Wl3, bl3, Wr3, br3, att3, bias3),
    )

    acc = None
    for i, (Wl, bl, Wr, br, att, bias) in enumerate(layers):
        R, d = Rs[i], ds[i]
        Wlp, blp = _pad_w(Wl, bl, R, ones_col=True)
        Wrp, brp = _pad_w(Wr, br, R, ones_col=False)
        attp = jnp.zeros((R,), jnp.float32).at[:d].set(att)
        if i == 0:
            xl, xr = _tc_prep(x, Wlp, blp, Wrp, brp, R)
        else:
            bias_prev = layers[i - 1][5].reshape(1, -1)
            xl, xr = _tc_comb_prep(acc, bias_prev, Wlp, blp, Wrp, brp,
                                   ds[i - 1], R)
        logits, wmax = _make_passA(R)(src, dst, xl, xr, attp)
        zeros = jnp.zeros((ZPER, R), jnp.float32)
        (acc,) = _make_passB(R)(src, dst, logits, wmax, xl, zeros)
        acc = acc.reshape(2, NROWS, R)

    return _tc_final(acc, bias3.reshape(1, -1), batch.reshape(N, 1).astype(jnp.int32),
                     W1, b1.reshape(1, -1), W2, b2.reshape(1, -1))


# speedup vs baseline: 11.5663x; 1.3091x over previous
import functools

import jax
import jax.numpy as jnp
from jax import lax
from jax.experimental import pallas as pl
from jax.experimental.pallas import tpu as pltpu
from jax.experimental.pallas import tpu_sc as plsc

N = 10000
N_GRAPHS = 256
PADROWS = 112
NROWS = N + PADROWS
ZPER = NROWS // 16
NW = 32
CHUNK = 128
E_TOT = 640000 + N
EPW = 20480
NCHUNK = EPW // CHUNK
E_PAD = NW * EPW

_mesh = plsc.VectorSubcoreMesh(core_axis_name="c", subcore_axis_name="s")

_GDN = lax.GatherDimensionNumbers(
    offset_dims=(), collapsed_slice_dims=(0,), start_index_map=(0,))


def _permute(v, idx):
    return lax.gather(v, idx[:, None], _GDN, slice_sizes=(1,),
                      mode=lax.GatherScatterMode.PROMISE_IN_BOUNDS)


def _vsum16(v, lane):
    for s in (8, 4, 2, 1):
        v = v + _permute(v, lane ^ s)
    return v


def _vmax16(v, lane):
    for s in (8, 4, 2, 1):
        v = jnp.maximum(v, _permute(v, lane ^ s))
    return v


def _tc_prep_body(h_ref, wl_ref, bl_ref, wr_ref, br_ref, xl_ref, xr_ref):
    h = h_ref[...]
    R = xl_ref.shape[1]
    pad = jnp.zeros((PADROWS, R), jnp.float32)
    xl = h @ wl_ref[...] + bl_ref[...]
    xr = h @ wr_ref[...] + br_ref[...]
    xl_ref[...] = jnp.concatenate([xl, pad], axis=0)
    xr_ref[...] = jnp.concatenate([xr, pad], axis=0)


def _tc_prep(h, Wlp, blp, Wrp, brp, R):
    return pl.pallas_call(
        _tc_prep_body,
        out_shape=[
            jax.ShapeDtypeStruct((NROWS, R), jnp.float32),
            jax.ShapeDtypeStruct((NROWS, R), jnp.float32),
        ],
    )(h, Wlp, blp, Wrp, brp)


def _combine(acc_ref, bias_ref, d):
    a = acc_ref[0] + acc_ref[1]
    num = lax.slice(a, (0, 0), (N, d))
    den = lax.slice(a, (0, d), (N, d + 1))
    return num / jnp.maximum(den, 1e-16) + bias_ref[...]


def _tc_comb_prep_body(acc_ref, bias_ref, wl_ref, bl_ref, wr_ref, br_ref,
                       xl_ref, xr_ref, *, d):
    h = jnp.maximum(_combine(acc_ref, bias_ref, d), 0.0)
    _tc_prep_body_inner(h, wl_ref, bl_ref, wr_ref, br_ref, xl_ref, xr_ref)


def _tc_prep_body_inner(h, wl_ref, bl_ref, wr_ref, br_ref, xl_ref, xr_ref):
    R = xl_ref.shape[1]
    pad = jnp.zeros((PADROWS, R), jnp.float32)
    xl = h @ wl_ref[...] + bl_ref[...]
    xr = h @ wr_ref[...] + br_ref[...]
    xl_ref[...] = jnp.concatenate([xl, pad], axis=0)
    xr_ref[...] = jnp.concatenate([xr, pad], axis=0)


def _tc_comb_prep(acc, bias, Wlp, blp, Wrp, brp, d, R):
    return pl.pallas_call(
        functools.partial(_tc_comb_prep_body, d=d),
        out_shape=[
            jax.ShapeDtypeStruct((NROWS, R), jnp.float32),
            jax.ShapeDtypeStruct((NROWS, R), jnp.float32),
        ],
    )(acc, bias, Wlp, blp, Wrp, brp)


def _tc_final_body(acc_ref, bias_ref, batch_ref, w1_ref, b1_ref, w2_ref,
                   b2_ref, o_ref):
    h = _combine(acc_ref, bias_ref, 30)
    seg = lax.broadcasted_iota(jnp.int32, (N, N_GRAPHS), 1)
    p = jnp.where(batch_ref[...] == seg, 1.0, 0.0)
    dn = (((0,), (0,)), ((), ()))
    pooled = lax.dot_general(p, h, dn, preferred_element_type=jnp.float32)
    counts = lax.dot_general(p, jnp.ones((N, 1), jnp.float32), dn,
                             preferred_element_type=jnp.float32)
    pooled = pooled / jnp.maximum(counts, 1.0)
    hid = jnp.maximum(pooled @ w1_ref[...] + b1_ref[...], 0.0)
    o_ref[...] = hid @ w2_ref[...] + b2_ref[...]


def _tc_final(acc, bias, batch_col, W1, b1, W2, b2):
    return pl.pallas_call(
        _tc_final_body,
        out_shape=jax.ShapeDtypeStruct((N_GRAPHS, 1), jnp.float32),
    )(acc, bias, batch_col, W1, b1, W2, b2)


def _make_passA(R, KV):

    @functools.partial(
        pl.kernel,
        mesh=_mesh,
        out_type=[
            jax.ShapeDtypeStruct((E_PAD,), jnp.float32),
            jax.ShapeDtypeStruct((NW, 16), jnp.float32),
        ],
        scratch_types=[
            pltpu.VMEM((CHUNK,), jnp.int32),
            pltpu.VMEM((CHUNK,), jnp.int32),
            pltpu.VMEM((CHUNK,), jnp.int32),
            pltpu.VMEM((CHUNK,), jnp.int32),
            pltpu.VMEM((CHUNK, R), jnp.float32),
            pltpu.VMEM((CHUNK, R), jnp.float32),
            pltpu.VMEM((CHUNK, R), jnp.float32),
            pltpu.VMEM((CHUNK, R), jnp.float32),
            pltpu.VMEM((CHUNK,), jnp.float32),
            pltpu.VMEM((16,), jnp.float32),
            pltpu.VMEM((R,), jnp.float32),
            pltpu.SemaphoreType.DMA,
            pltpu.SemaphoreType.DMA,
            pltpu.SemaphoreType.DMA,
            pltpu.SemaphoreType.DMA,
        ],
    )
    def passA(src_hbm, dst_hbm, xl_hbm, xr_hbm, att_hbm, logits_hbm,
              wmax_hbm, srcv0, dstv0, srcv1, dstv1, rowsA0, rowsB0, rowsA1,
              rowsB1, lv, mv, attv, semA0, semB0, semA1, semB1):
        wid = lax.axis_index("s") * 2 + lax.axis_index("c")
        pltpu.sync_copy(att_hbm, attv)
        att_regs = [attv[pl.ds(16 * k, 16)] for k in range(KV)]
        lane = lax.broadcasted_iota(jnp.int32, (16,), 0)

        sets = ((srcv0, dstv0, rowsA0, rowsB0, semA0, semB0),
                (srcv1, dstv1, rowsA1, rowsB1, semA1, semB1))

        def fetch(c, s):
            sv, dv, ra, rb, sa, sb = sets[s]
            base = wid * EPW + c * CHUNK
            pltpu.sync_copy(src_hbm.at[pl.ds(base, CHUNK)], sv)
            pltpu.sync_copy(dst_hbm.at[pl.ds(base, CHUNK)], dv)
            cpa = pltpu.async_copy(xl_hbm.at[sv], ra, sa)
            cpb = pltpu.async_copy(xr_hbm.at[dv], rb, sb)
            return cpa, cpb

        def compute(c, s, smv):
            _, _, ra, rb, _, _ = sets[s]

            def group_body(t, sm):
                lgvec = jnp.zeros((16,), jnp.float32)
                for jj in range(16):
                    j = 16 * t + jj
                    acc = jnp.zeros((16,), jnp.float32)
                    for k in range(KV):
                        u = (ra[j, pl.ds(16 * k, 16)]
                             + rb[j, pl.ds(16 * k, 16)])
                        e = jnp.maximum(u, 0.2 * u)
                        acc = acc + e * att_regs[k]
                    lg = _vsum16(acc, lane)
                    lgvec = jnp.where(lane == jj, lg, lgvec)
                lv[pl.ds(16 * t, 16)] = lgvec
                return jnp.maximum(sm, lgvec)

            smv = lax.fori_loop(0, CHUNK // 16, group_body, smv)
            base = wid * EPW + c * CHUNK
            pltpu.sync_copy(lv, logits_hbm.at[pl.ds(base, CHUNK)])
            return smv

        def body(i, smv):
            c0 = 2 * i
            cpa0, cpb0 = fetch(c0, 0)
            cpa1, cpb1 = fetch(c0 + 1, 1)
            cpa0.wait()
            cpb0.wait()
            smv = compute(c0, 0, smv)
            cpa1.wait()
            cpb1.wait()
            return compute(c0 + 1, 1, smv)

        smaxvec = lax.fori_loop(0, NCHUNK // 2, body,
                                jnp.full((16,), -1e30, jnp.float32))
        mv[...] = smaxvec
        pltpu.sync_copy(mv, wmax_hbm.at[wid])

    return passA


def _make_passB(R, KV):

    @functools.partial(
        pl.kernel,
        mesh=_mesh,
        out_type=[jax.ShapeDtypeStruct((2 * NROWS, R), jnp.float32)],
        scratch_types=[
            pltpu.VMEM((CHUNK,), jnp.int32),
            pltpu.VMEM((CHUNK,), jnp.int32),
            pltpu.VMEM((CHUNK,), jnp.int32),
            pltpu.VMEM((CHUNK,), jnp.int32),
            pltpu.VMEM((CHUNK, R), jnp.float32),
            pltpu.VMEM((CHUNK, R), jnp.float32),
            pltpu.VMEM((CHUNK,), jnp.float32),
            pltpu.VMEM((CHUNK,), jnp.float32),
            pltpu.VMEM((NW, 16), jnp.float32),
            pltpu.VMEM_SHARED((NROWS, R), jnp.float32),
            pltpu.SemaphoreType.DMA,
            pltpu.SemaphoreType.DMA,
            pltpu.SemaphoreType.DMA,
            pltpu.SemaphoreType.DMA,
        ],
    )
    def passB(src_hbm, dst_hbm, logits_hbm, wmax_hbm, xl_hbm, zeros_hbm,
              acc_hbm, srcv0, dstv0, srcv1, dstv1, rows0, rows1, lv0, lv1,
              mw, acc_sh, semA0, semA1, semS0, semS1):
        cid = lax.axis_index("c")
        sid = lax.axis_index("s")
        wid = sid * 2 + cid
        lane = lax.broadcasted_iota(jnp.int32, (16,), 0)
        pltpu.sync_copy(wmax_hbm, mw)
        mvec = mw[0, pl.ds(0, 16)]
        for k in range(1, NW):
            mvec = jnp.maximum(mvec, mw[k, pl.ds(0, 16)])
        mfull = _vmax16(mvec, lane)

        pltpu.sync_copy(zeros_hbm, acc_sh.at[pl.ds(sid * ZPER, ZPER)])
        plsc.subcore_barrier()

        sets = ((srcv0, dstv0, rows0, lv0, semA0, semS0),
                (srcv1, dstv1, rows1, lv1, semA1, semS1))

        def fetch(c, s):
            sv, dv, ro, lv, sa, _ = sets[s]
            base = wid * EPW + c * CHUNK
            pltpu.sync_copy(src_hbm.at[pl.ds(base, CHUNK)], sv)
            pltpu.sync_copy(dst_hbm.at[pl.ds(base, CHUNK)], dv)
            pltpu.sync_copy(logits_hbm.at[pl.ds(base, CHUNK)], lv)
            return pltpu.async_copy(xl_hbm.at[sv], ro, sa)

        def scale_scatter(s):
            sv, dv, ro, lv, sa, ss = sets[s]
            for t in range(CHUNK // 16):
                lv[pl.ds(16 * t, 16)] = jnp.exp(lv[pl.ds(16 * t, 16)] - mfull)

            def group_body(t, _):
                exvec = lv[pl.ds(16 * t, 16)]
                for jj in range(16):
                    j = 16 * t + jj
                    sc = _permute(exvec, jnp.full((16,), jj, jnp.int32))
                    for k in range(KV):
                        ro[j, pl.ds(16 * k, 16)] = (
                            ro[j, pl.ds(16 * k, 16)] * sc)
                return 0

            lax.fori_loop(0, CHUNK // 16, group_body, 0)
            return pltpu.async_copy(ro, acc_sh.at[dv], ss, add=True)

        def body(i, carry):
            c0 = 2 * i
            cpa0 = fetch(c0, 0)
            cpa1 = fetch(c0 + 1, 1)
            cpa0.wait()
            cps0 = scale_scatter(0)
            cpa1.wait()
            cps1 = scale_scatter(1)
            cps0.wait()
            cps1.wait()
            return carry

        lax.fori_loop(0, NCHUNK // 2, body, 0)
        plsc.subcore_barrier()
        pltpu.sync_copy(
            acc_sh.at[pl.ds(sid * ZPER, ZPER)],
            acc_hbm.at[pl.ds(cid * NROWS + sid * ZPER, ZPER)],
        )

    return passB


def _pad_w(W, b, R, ones_col):
    din, d = W.shape
    Wp = jnp.zeros((din, R), jnp.float32).at[:, :d].set(W)
    bp = jnp.zeros((R,), jnp.float32).at[:d].set(b)
    if ones_col:
        bp = bp.at[d].set(1.0)
    return Wp, bp.reshape(1, R)


def kernel(x, edge_index, batch, Wl1, bl1, Wr1, br1, att1, bias1, Wl2, bl2,
           Wr2, br2, att2, bias2, Wl3, bl3, Wr3, br3, att3, bias3, W1, b1,
           W2, b2):
    loop = jnp.arange(N, dtype=edge_index.dtype)
    fill = jnp.full((E_PAD - E_TOT,), N, jnp.int32)
    src = jnp.concatenate([edge_index[0], loop, fill])
    dst = jnp.concatenate([edge_index[1], loop, fill])

    Rs = (128, 128, 128)
    ds = (100, 60, 30)
    layers = (
        (Wl1, bl1, Wr1, br1, att1, bias1),
        (Wl2, bl2, Wr2, br2, att2, bias2),
        (Wl3, bl3, Wr3, br3, att3, bias3),
    )

    acc = None
    for i, (Wl, bl, Wr, br, att, bias) in enumerate(layers):
        R, d = Rs[i], ds[i]
        Wlp, blp = _pad_w(Wl, bl, R, ones_col=True)
        Wrp, brp = _pad_w(Wr, br, R, ones_col=False)
        attp = jnp.zeros((R,), jnp.float32).at[:d].set(att)
        if i == 0:
            xl, xr = _tc_prep(x, Wlp, blp, Wrp, brp, R)
        else:
            bias_prev = layers[i - 1][5].reshape(1, -1)
            xl, xr = _tc_comb_prep(acc, bias_prev, Wlp, blp, Wrp, brp,
                                   ds[i - 1], R)
        KV = (d + 16) // 16
        logits, wmax = _make_passA(R, KV)(src, dst, xl, xr, attp)
        zeros = jnp.zeros((ZPER, R), jnp.float32)
        (acc,) = _make_passB(R, KV)(src, dst, logits, wmax, xl, zeros)
        acc = acc.reshape(2, NROWS, R)

    return _tc_final(acc, bias3.reshape(1, -1), batch.reshape(N, 1).astype(jnp.int32),
                     W1, b1.reshape(1, -1), W2, b2.reshape(1, -1))


# speedup vs baseline: 11.6596x; 1.0081x over previous
import functools

import jax
import jax.numpy as jnp
from jax import lax
from jax.experimental import pallas as pl
from jax.experimental.pallas import tpu as pltpu
from jax.experimental.pallas import tpu_sc as plsc

N = 10000
N_GRAPHS = 256
PADROWS = 112
NROWS = N + PADROWS
ZPER = NROWS // 16
NW = 32
CHUNK = 128
E_TOT = 640000 + N
EPW = 20480
NCHUNK = EPW // CHUNK
E_PAD = NW * EPW

_mesh = plsc.VectorSubcoreMesh(core_axis_name="c", subcore_axis_name="s")

_GDN = lax.GatherDimensionNumbers(
    offset_dims=(), collapsed_slice_dims=(0,), start_index_map=(0,))


def _permute(v, idx):
    return lax.gather(v, idx[:, None], _GDN, slice_sizes=(1,),
                      mode=lax.GatherScatterMode.PROMISE_IN_BOUNDS)


def _vsum16(v, lane):
    for s in (8, 4, 2, 1):
        v = v + _permute(v, lane ^ s)
    return v


def _vmax16(v, lane):
    for s in (8, 4, 2, 1):
        v = jnp.maximum(v, _permute(v, lane ^ s))
    return v


def _tc_prep_body(h_ref, wl_ref, bl_ref, wr_ref, br_ref, xl_ref, xr_ref):
    h = h_ref[...]
    R = xl_ref.shape[1]
    pad = jnp.zeros((PADROWS, R), jnp.float32)
    xl = h @ wl_ref[...] + bl_ref[...]
    xr = h @ wr_ref[...] + br_ref[...]
    xl_ref[...] = jnp.concatenate([xl, pad], axis=0)
    xr_ref[...] = jnp.concatenate([xr, pad], axis=0)


def _tc_prep(h, Wlp, blp, Wrp, brp, R):
    return pl.pallas_call(
        _tc_prep_body,
        out_shape=[
            jax.ShapeDtypeStruct((NROWS, R), jnp.float32),
            jax.ShapeDtypeStruct((NROWS, R), jnp.float32),
        ],
    )(h, Wlp, blp, Wrp, brp)


def _combine(acc_ref, bias_ref, d):
    a = acc_ref[0] + acc_ref[1]
    num = lax.slice(a, (0, 0), (N, d))
    den = lax.slice(a, (0, d), (N, d + 1))
    return num / jnp.maximum(den, 1e-16) + bias_ref[...]


def _tc_comb_prep_body(acc_ref, bias_ref, wl_ref, bl_ref, wr_ref, br_ref,
                       xl_ref, xr_ref, *, d):
    h = jnp.maximum(_combine(acc_ref, bias_ref, d), 0.0)
    _tc_prep_body_inner(h, wl_ref, bl_ref, wr_ref, br_ref, xl_ref, xr_ref)


def _tc_prep_body_inner(h, wl_ref, bl_ref, wr_ref, br_ref, xl_ref, xr_ref):
    R = xl_ref.shape[1]
    pad = jnp.zeros((PADROWS, R), jnp.float32)
    xl = h @ wl_ref[...] + bl_ref[...]
    xr = h @ wr_ref[...] + br_ref[...]
    xl_ref[...] = jnp.concatenate([xl, pad], axis=0)
    xr_ref[...] = jnp.concatenate([xr, pad], axis=0)


def _tc_comb_prep(acc, bias, Wlp, blp, Wrp, brp, d, R):
    return pl.pallas_call(
        functools.partial(_tc_comb_prep_body, d=d),
        out_shape=[
            jax.ShapeDtypeStruct((NROWS, R), jnp.float32),
            jax.ShapeDtypeStruct((NROWS, R), jnp.float32),
        ],
    )(acc, bias, Wlp, blp, Wrp, brp)


def _tc_final_body(acc_ref, bias_ref, batch_ref, w1_ref, b1_ref, w2_ref,
                   b2_ref, o_ref):
    h = _combine(acc_ref, bias_ref, 30)
    seg = lax.broadcasted_iota(jnp.int32, (N, N_GRAPHS), 1)
    p = jnp.where(batch_ref[...] == seg, 1.0, 0.0)
    dn = (((0,), (0,)), ((), ()))
    pooled = lax.dot_general(p, h, dn, preferred_element_type=jnp.float32)
    counts = lax.dot_general(p, jnp.ones((N, 1), jnp.float32), dn,
                             preferred_element_type=jnp.float32)
    pooled = pooled / jnp.maximum(counts, 1.0)
    hid = jnp.maximum(pooled @ w1_ref[...] + b1_ref[...], 0.0)
    o_ref[...] = hid @ w2_ref[...] + b2_ref[...]


def _tc_final(acc, bias, batch_col, W1, b1, W2, b2):
    return pl.pallas_call(
        _tc_final_body,
        out_shape=jax.ShapeDtypeStruct((N_GRAPHS, 1), jnp.float32),
    )(acc, bias, batch_col, W1, b1, W2, b2)


def _make_passA(R, KV):

    @functools.partial(
        pl.kernel,
        mesh=_mesh,
        out_type=[
            jax.ShapeDtypeStruct((E_PAD,), jnp.float32),
            jax.ShapeDtypeStruct((NW, 16), jnp.float32),
        ],
        scratch_types=[
            pltpu.VMEM((NCHUNK, CHUNK), jnp.int32),
            pltpu.VMEM((NCHUNK, CHUNK), jnp.int32),
            pltpu.VMEM((CHUNK,), jnp.float32),
            pltpu.VMEM((CHUNK, R), jnp.float32),
            pltpu.VMEM((CHUNK, R), jnp.float32),
            pltpu.VMEM((CHUNK, R), jnp.float32),
            pltpu.VMEM((CHUNK, R), jnp.float32),
            pltpu.VMEM((16,), jnp.float32),
            pltpu.VMEM((R,), jnp.float32),
            pltpu.SemaphoreType.DMA,
            pltpu.SemaphoreType.DMA,
            pltpu.SemaphoreType.DMA,
            pltpu.SemaphoreType.DMA,
        ],
    )
    def passA(src_hbm, dst_hbm, xl_hbm, xr_hbm, att_hbm, logits_hbm,
              wmax_hbm, srcall, dstall, lv, rowsA0, rowsB0, rowsA1,
              rowsB1, mv, attv, semA0, semB0, semA1, semB1):
        wid = lax.axis_index("s") * 2 + lax.axis_index("c")
        pltpu.sync_copy(att_hbm, attv)
        pltpu.sync_copy(src_hbm.at[pl.ds(wid * NCHUNK, NCHUNK)], srcall)
        pltpu.sync_copy(dst_hbm.at[pl.ds(wid * NCHUNK, NCHUNK)], dstall)
        att_regs = [attv[pl.ds(16 * k, 16)] for k in range(KV)]
        lane = lax.broadcasted_iota(jnp.int32, (16,), 0)

        sets = ((rowsA0, rowsB0, semA0, semB0),
                (rowsA1, rowsB1, semA1, semB1))

        def fetch(c, s):
            ra, rb, sa, sb = sets[s]
            cpa = pltpu.async_copy(xl_hbm.at[srcall.at[c]], ra, sa)
            cpb = pltpu.async_copy(xr_hbm.at[dstall.at[c]], rb, sb)
            return cpa, cpb

        def compute(c, s, smv):
            ra, rb, _, _ = sets[s]

            def group_body(t, sm):
                lgvec = jnp.zeros((16,), jnp.float32)
                for jj in range(16):
                    j = 16 * t + jj
                    acc = jnp.zeros((16,), jnp.float32)
                    for k in range(KV):
                        u = (ra[j, pl.ds(16 * k, 16)]
                             + rb[j, pl.ds(16 * k, 16)])
                        e = jnp.maximum(u, 0.2 * u)
                        acc = acc + e * att_regs[k]
                    lg = _vsum16(acc, lane)
                    lgvec = jnp.where(lane == jj, lg, lgvec)
                lv[pl.ds(16 * t, 16)] = lgvec
                return jnp.maximum(sm, lgvec)

            smv = lax.fori_loop(0, CHUNK // 16, group_body, smv)
            pltpu.sync_copy(lv, logits_hbm.at[pl.ds(wid * EPW + c * CHUNK,
                                                    CHUNK)])
            return smv

        def body(i, smv):
            c0 = 2 * i
            cpa0, cpb0 = fetch(c0, 0)
            cpa1, cpb1 = fetch(c0 + 1, 1)
            cpa0.wait()
            cpb0.wait()
            smv = compute(c0, 0, smv)
            cpa1.wait()
            cpb1.wait()
            return compute(c0 + 1, 1, smv)

        smaxvec = lax.fori_loop(0, NCHUNK // 2, body,
                                jnp.full((16,), -1e30, jnp.float32))
        mv[...] = smaxvec
        pltpu.sync_copy(mv, wmax_hbm.at[wid])

    return passA


def _make_passB(R, KV):

    @functools.partial(
        pl.kernel,
        mesh=_mesh,
        out_type=[jax.ShapeDtypeStruct((2 * NROWS, R), jnp.float32)],
        scratch_types=[
            pltpu.VMEM((CHUNK,), jnp.int32),
            pltpu.VMEM((CHUNK,), jnp.int32),
            pltpu.VMEM((CHUNK,), jnp.int32),
            pltpu.VMEM((CHUNK,), jnp.int32),
            pltpu.VMEM((CHUNK, R), jnp.float32),
            pltpu.VMEM((CHUNK, R), jnp.float32),
            pltpu.VMEM((CHUNK,), jnp.float32),
            pltpu.VMEM((CHUNK,), jnp.float32),
            pltpu.VMEM((NW, 16), jnp.float32),
            pltpu.VMEM_SHARED((NROWS, R), jnp.float32),
            pltpu.SemaphoreType.DMA,
            pltpu.SemaphoreType.DMA,
            pltpu.SemaphoreType.DMA,
            pltpu.SemaphoreType.DMA,
        ],
    )
    def passB(src_hbm, dst_hbm, logits_hbm, wmax_hbm, xl_hbm, zeros_hbm,
              acc_hbm, srcv0, dstv0, srcv1, dstv1, rows0, rows1, lv0, lv1,
              mw, acc_sh, semA0, semA1, semS0, semS1):
        cid = lax.axis_index("c")
        sid = lax.axis_index("s")
        wid = sid * 2 + cid
        lane = lax.broadcasted_iota(jnp.int32, (16,), 0)
        pltpu.sync_copy(wmax_hbm, mw)
        mvec = mw[0, pl.ds(0, 16)]
        for k in range(1, NW):
            mvec = jnp.maximum(mvec, mw[k, pl.ds(0, 16)])
        mfull = _vmax16(mvec, lane)

        pltpu.sync_copy(zeros_hbm, acc_sh.at[pl.ds(sid * ZPER, ZPER)])
        plsc.subcore_barrier()

        sets = ((srcv0, dstv0, rows0, lv0, semA0, semS0),
                (srcv1, dstv1, rows1, lv1, semA1, semS1))

        def fetch(c, s):
            sv, dv, ro, lv, sa, _ = sets[s]
            base = wid * EPW + c * CHUNK
            pltpu.sync_copy(src_hbm.at[pl.ds(base, CHUNK)], sv)
            pltpu.sync_copy(dst_hbm.at[pl.ds(base, CHUNK)], dv)
            pltpu.sync_copy(logits_hbm.at[pl.ds(base, CHUNK)], lv)
            return pltpu.async_copy(xl_hbm.at[sv], ro, sa)

        def scale_scatter(s):
            sv, dv, ro, lv, sa, ss = sets[s]
            for t in range(CHUNK // 16):
                lv[pl.ds(16 * t, 16)] = jnp.exp(lv[pl.ds(16 * t, 16)]
                                                - mfull)

            def group_body(t, _):
                exvec = lv[pl.ds(16 * t, 16)]
                for jj in range(16):
                    j = 16 * t + jj
                    sc = _permute(exvec, jnp.full((16,), jj, jnp.int32))
                    for k in range(KV):
                        ro[j, pl.ds(16 * k, 16)] = (
                            ro[j, pl.ds(16 * k, 16)] * sc)
                return 0

            lax.fori_loop(0, CHUNK // 16, group_body, 0)
            return pltpu.async_copy(ro, acc_sh.at[dv], ss, add=True)

        def body(i, carry):
            c0 = 2 * i
            cpa0 = fetch(c0, 0)
            cpa1 = fetch(c0 + 1, 1)
            cpa0.wait()
            cps0 = scale_scatter(0)
            cpa1.wait()
            cps1 = scale_scatter(1)
            cps0.wait()
            cps1.wait()
            return carry

        lax.fori_loop(0, NCHUNK // 2, body, 0)
        plsc.subcore_barrier()
        pltpu.sync_copy(
            acc_sh.at[pl.ds(sid * ZPER, ZPER)],
            acc_hbm.at[pl.ds(cid * NROWS + sid * ZPER, ZPER)],
        )

    return passB


def _pad_w(W, b, R, ones_col):
    din, d = W.shape
    Wp = jnp.zeros((din, R), jnp.float32).at[:, :d].set(W)
    bp = jnp.zeros((R,), jnp.float32).at[:d].set(b)
    if ones_col:
        bp = bp.at[d].set(1.0)
    return Wp, bp.reshape(1, R)


def kernel(x, edge_index, batch, Wl1, bl1, Wr1, br1, att1, bias1, Wl2, bl2,
           Wr2, br2, att2, bias2, Wl3, bl3, Wr3, br3, att3, bias3, W1, b1,
           W2, b2):
    loop = jnp.arange(N, dtype=edge_index.dtype)
    fill = jnp.full((E_PAD - E_TOT,), N, jnp.int32)
    src1 = jnp.concatenate([edge_index[0], loop, fill])
    dst1 = jnp.concatenate([edge_index[1], loop, fill])
    src2 = src1.reshape(NW * NCHUNK, CHUNK)
    dst2 = dst1.reshape(NW * NCHUNK, CHUNK)

    Rs = (128, 128, 128)
    ds = (100, 60, 30)
    layers = (
        (Wl1, bl1, Wr1, br1, att1, bias1),
        (Wl2, bl2, Wr2, br2, att2, bias2),
        (Wl3, bl3, Wr3, br3, att3, bias3),
    )

    acc = None
    for i, (Wl, bl, Wr, br, att, bias) in enumerate(layers):
        R, d = Rs[i], ds[i]
        Wlp, blp = _pad_w(Wl, bl, R, ones_col=True)
        Wrp, brp = _pad_w(Wr, br, R, ones_col=False)
        attp = jnp.zeros((R,), jnp.float32).at[:d].set(att)
        if i == 0:
            xl, xr = _tc_prep(x, Wlp, blp, Wrp, brp, R)
        else:
            bias_prev = layers[i - 1][5].reshape(1, -1)
            xl, xr = _tc_comb_prep(acc, bias_prev, Wlp, blp, Wrp, brp,
                                   ds[i - 1], R)
        KV = (d + 16) // 16
        logits, wmax = _make_passA(R, KV)(src2, dst2, xl, xr, attp)
        zeros = jnp.zeros((ZPER, R), jnp.float32)
        (acc,) = _make_passB(R, KV)(src1, dst1, logits, wmax, xl, zeros)
        acc = acc.reshape(2, NROWS, R)

    return _tc_final(acc, bias3.reshape(1, -1), batch.reshape(N, 1).astype(jnp.int32),
                     W1, b1.reshape(1, -1), W2, b2.reshape(1, -1))


# speedup vs baseline: 13.7816x; 1.1820x over previous
import functools

import jax
import jax.numpy as jnp
from jax import lax
from jax.experimental import pallas as pl
from jax.experimental.pallas import tpu as pltpu
from jax.experimental.pallas import tpu_sc as plsc

N = 10000
N_GRAPHS = 256
PADROWS = 112
NROWS = N + PADROWS
ZPER = NROWS // 16
NW = 32
CHUNK = 128
E_TOT = 640000 + N
EPW = 20480
NCHUNK = EPW // CHUNK
C0 = 192
C1 = 2 * NCHUNK - C0
CMAX = max(C0, C1)
SLACK = CMAX - min(C0, C1)
E_PAD = NW * EPW

_mesh = plsc.VectorSubcoreMesh(core_axis_name="c", subcore_axis_name="s")

_GDN = lax.GatherDimensionNumbers(
    offset_dims=(), collapsed_slice_dims=(0,), start_index_map=(0,))


def _permute(v, idx):
    return lax.gather(v, idx[:, None], _GDN, slice_sizes=(1,),
                      mode=lax.GatherScatterMode.PROMISE_IN_BOUNDS)


def _vsum16(v, lane):
    for s in (8, 4, 2, 1):
        v = v + _permute(v, lane ^ s)
    return v


def _vmax16(v, lane):
    for s in (8, 4, 2, 1):
        v = jnp.maximum(v, _permute(v, lane ^ s))
    return v


def _tc_prep_body(h_ref, wl_ref, bl_ref, wr_ref, br_ref, xl_ref, xr_ref):
    h = h_ref[...]
    R = xl_ref.shape[1]
    pad = jnp.zeros((PADROWS, R), jnp.float32)
    xl = h @ wl_ref[...] + bl_ref[...]
    xr = h @ wr_ref[...] + br_ref[...]
    xl_ref[...] = jnp.concatenate([xl, pad], axis=0)
    xr_ref[...] = jnp.concatenate([xr, pad], axis=0)


def _tc_prep(h, Wlp, blp, Wrp, brp, R):
    return pl.pallas_call(
        _tc_prep_body,
        out_shape=[
            jax.ShapeDtypeStruct((NROWS, R), jnp.float32),
            jax.ShapeDtypeStruct((NROWS, R), jnp.float32),
        ],
    )(h, Wlp, blp, Wrp, brp)


def _combine(acc_ref, bias_ref, d):
    a = acc_ref[0] + acc_ref[1]
    num = lax.slice(a, (0, 0), (N, d))
    den = lax.slice(a, (0, d), (N, d + 1))
    return num / jnp.maximum(den, 1e-16) + bias_ref[...]


def _tc_comb_prep_body(acc_ref, bias_ref, wl_ref, bl_ref, wr_ref, br_ref,
                       xl_ref, xr_ref, *, d):
    h = jnp.maximum(_combine(acc_ref, bias_ref, d), 0.0)
    _tc_prep_body_inner(h, wl_ref, bl_ref, wr_ref, br_ref, xl_ref, xr_ref)


def _tc_prep_body_inner(h, wl_ref, bl_ref, wr_ref, br_ref, xl_ref, xr_ref):
    R = xl_ref.shape[1]
    pad = jnp.zeros((PADROWS, R), jnp.float32)
    xl = h @ wl_ref[...] + bl_ref[...]
    xr = h @ wr_ref[...] + br_ref[...]
    xl_ref[...] = jnp.concatenate([xl, pad], axis=0)
    xr_ref[...] = jnp.concatenate([xr, pad], axis=0)


def _tc_comb_prep(acc, bias, Wlp, blp, Wrp, brp, d, R):
    return pl.pallas_call(
        functools.partial(_tc_comb_prep_body, d=d),
        out_shape=[
            jax.ShapeDtypeStruct((NROWS, R), jnp.float32),
            jax.ShapeDtypeStruct((NROWS, R), jnp.float32),
        ],
    )(acc, bias, Wlp, blp, Wrp, brp)


def _tc_final_body(acc_ref, bias_ref, batch_ref, w1_ref, b1_ref, w2_ref,
                   b2_ref, o_ref):
    h = _combine(acc_ref, bias_ref, 30)
    seg = lax.broadcasted_iota(jnp.int32, (N, N_GRAPHS), 1)
    p = jnp.where(batch_ref[...] == seg, 1.0, 0.0)
    dn = (((0,), (0,)), ((), ()))
    pooled = lax.dot_general(p, h, dn, preferred_element_type=jnp.float32)
    counts = lax.dot_general(p, jnp.ones((N, 1), jnp.float32), dn,
                             preferred_element_type=jnp.float32)
    pooled = pooled / jnp.maximum(counts, 1.0)
    hid = jnp.maximum(pooled @ w1_ref[...] + b1_ref[...], 0.0)
    o_ref[...] = hid @ w2_ref[...] + b2_ref[...]


def _tc_final(acc, bias, batch_col, W1, b1, W2, b2):
    return pl.pallas_call(
        _tc_final_body,
        out_shape=jax.ShapeDtypeStruct((N_GRAPHS, 1), jnp.float32),
    )(acc, bias, batch_col, W1, b1, W2, b2)


def _make_passA(R, KV):

    @functools.partial(
        pl.kernel,
        mesh=_mesh,
        out_type=[
            jax.ShapeDtypeStruct((E_PAD + SLACK * CHUNK,), jnp.float32),
            jax.ShapeDtypeStruct((NW, 16), jnp.float32),
        ],
        scratch_types=[
            pltpu.VMEM((CMAX, CHUNK), jnp.int32),
            pltpu.VMEM((CMAX, CHUNK), jnp.int32),
            pltpu.VMEM((CHUNK,), jnp.float32),
            pltpu.VMEM((CHUNK, R), jnp.float32),
            pltpu.VMEM((CHUNK, R), jnp.float32),
            pltpu.VMEM((CHUNK, R), jnp.float32),
            pltpu.VMEM((CHUNK, R), jnp.float32),
            pltpu.VMEM((16,), jnp.float32),
            pltpu.VMEM((R,), jnp.float32),
            pltpu.SemaphoreType.DMA,
            pltpu.SemaphoreType.DMA,
            pltpu.SemaphoreType.DMA,
            pltpu.SemaphoreType.DMA,
        ],
    )
    def passA(src_hbm, dst_hbm, xl_hbm, xr_hbm, att_hbm, logits_hbm,
              wmax_hbm, srcall, dstall, lv, rowsA0, rowsB0, rowsA1,
              rowsB1, mv, attv, semA0, semB0, semA1, semB1):
        cid = lax.axis_index("c")
        sid = lax.axis_index("s")
        wid = sid * 2 + cid
        row0 = sid * (2 * NCHUNK) + jnp.where(cid == 0, 0, C0)
        nch2 = jnp.where(cid == 0, C0 // 2, C1 // 2)
        pltpu.sync_copy(att_hbm, attv)
        pltpu.sync_copy(src_hbm.at[pl.ds(row0, CMAX)], srcall)
        pltpu.sync_copy(dst_hbm.at[pl.ds(row0, CMAX)], dstall)
        att_regs = [attv[pl.ds(16 * k, 16)] for k in range(KV)]
        lane = lax.broadcasted_iota(jnp.int32, (16,), 0)

        sets = ((rowsA0, rowsB0, semA0, semB0),
                (rowsA1, rowsB1, semA1, semB1))

        def fetch(c, s):
            ra, rb, sa, sb = sets[s]
            cpa = pltpu.async_copy(xl_hbm.at[srcall.at[c]], ra, sa)
            cpb = pltpu.async_copy(xr_hbm.at[dstall.at[c]], rb, sb)
            return cpa, cpb

        def compute(c, s, smv):
            ra, rb, _, _ = sets[s]

            def group_body(t, sm):
                lgvec = jnp.zeros((16,), jnp.float32)
                for jj in range(16):
                    j = 16 * t + jj
                    acc = jnp.zeros((16,), jnp.float32)
                    for k in range(KV):
                        u = (ra[j, pl.ds(16 * k, 16)]
                             + rb[j, pl.ds(16 * k, 16)])
                        e = jnp.maximum(u, 0.2 * u)
                        acc = acc + e * att_regs[k]
                    lg = _vsum16(acc, lane)
                    lgvec = jnp.where(lane == jj, lg, lgvec)
                lv[pl.ds(16 * t, 16)] = lgvec
                return jnp.maximum(sm, lgvec)

            smv = lax.fori_loop(0, CHUNK // 16, group_body, smv)
            pltpu.sync_copy(
                lv, logits_hbm.at[pl.ds((row0 + c) * CHUNK, CHUNK)])
            return smv

        def body(i, smv):
            c0 = 2 * i
            cpa0, cpb0 = fetch(c0, 0)
            cpa1, cpb1 = fetch(c0 + 1, 1)
            cpa0.wait()
            cpb0.wait()
            smv = compute(c0, 0, smv)
            cpa1.wait()
            cpb1.wait()
            return compute(c0 + 1, 1, smv)

        smaxvec = lax.fori_loop(0, nch2, body,
                                jnp.full((16,), -1e30, jnp.float32))
        mv[...] = smaxvec
        pltpu.sync_copy(mv, wmax_hbm.at[wid])

    return passA


def _make_passB(R, KV):

    @functools.partial(
        pl.kernel,
        mesh=_mesh,
        out_type=[jax.ShapeDtypeStruct((2 * NROWS, R), jnp.float32)],
        scratch_types=[
            pltpu.VMEM((CHUNK,), jnp.int32),
            pltpu.VMEM((CHUNK,), jnp.int32),
            pltpu.VMEM((CHUNK,), jnp.int32),
            pltpu.VMEM((CHUNK,), jnp.int32),
            pltpu.VMEM((CHUNK, R), jnp.float32),
            pltpu.VMEM((CHUNK, R), jnp.float32),
            pltpu.VMEM((CHUNK,), jnp.float32),
            pltpu.VMEM((CHUNK,), jnp.float32),
            pltpu.VMEM((NW, 16), jnp.float32),
            pltpu.VMEM_SHARED((NROWS, R), jnp.float32),
            pltpu.SemaphoreType.DMA,
            pltpu.SemaphoreType.DMA,
            pltpu.SemaphoreType.DMA,
            pltpu.SemaphoreType.DMA,
        ],
    )
    def passB(src_hbm, dst_hbm, logits_hbm, wmax_hbm, xl_hbm, zeros_hbm,
              acc_hbm, srcv0, dstv0, srcv1, dstv1, rows0, rows1, lv0, lv1,
              mw, acc_sh, semA0, semA1, semS0, semS1):
        cid = lax.axis_index("c")
        sid = lax.axis_index("s")
        row0 = sid * (2 * NCHUNK) + jnp.where(cid == 0, 0, C0)
        nch2 = jnp.where(cid == 0, C0 // 2, C1 // 2)
        lane = lax.broadcasted_iota(jnp.int32, (16,), 0)
        pltpu.sync_copy(wmax_hbm, mw)
        mvec = mw[0, pl.ds(0, 16)]
        for k in range(1, NW):
            mvec = jnp.maximum(mvec, mw[k, pl.ds(0, 16)])
        mfull = _vmax16(mvec, lane)

        pltpu.sync_copy(zeros_hbm, acc_sh.at[pl.ds(sid * ZPER, ZPER)])
        plsc.subcore_barrier()

        sets = ((srcv0, dstv0, rows0, lv0, semA0, semS0),
                (srcv1, dstv1, rows1, lv1, semA1, semS1))

        def fetch(c, s):
            sv, dv, ro, lv, sa, _ = sets[s]
            base = (row0 + c) * CHUNK
            pltpu.sync_copy(src_hbm.at[pl.ds(base, CHUNK)], sv)
            pltpu.sync_copy(dst_hbm.at[pl.ds(base, CHUNK)], dv)
            pltpu.sync_copy(logits_hbm.at[pl.ds(base, CHUNK)], lv)
            return pltpu.async_copy(xl_hbm.at[sv], ro, sa)

        def scale_scatter(s):
            sv, dv, ro, lv, sa, ss = sets[s]
            for t in range(CHUNK // 16):
                lv[pl.ds(16 * t, 16)] = jnp.exp(lv[pl.ds(16 * t, 16)]
                                                - mfull)

            def group_body(t, _):
                exvec = lv[pl.ds(16 * t, 16)]
                for jj in range(16):
                    j = 16 * t + jj
                    sc = _permute(exvec, jnp.full((16,), jj, jnp.int32))
                    for k in range(KV):
                        ro[j, pl.ds(16 * k, 16)] = (
                            ro[j, pl.ds(16 * k, 16)] * sc)
                return 0

            lax.fori_loop(0, CHUNK // 16, group_body, 0)
            return pltpu.async_copy(ro, acc_sh.at[dv], ss, add=True)

        def body(i, carry):
            c0 = 2 * i
            cpa0 = fetch(c0, 0)
            cpa1 = fetch(c0 + 1, 1)
            cpa0.wait()
            cps0 = scale_scatter(0)
            cpa1.wait()
            cps1 = scale_scatter(1)
            cps0.wait()
            cps1.wait()
            return carry

        lax.fori_loop(0, nch2, body, 0)
        plsc.subcore_barrier()
        pltpu.sync_copy(
            acc_sh.at[pl.ds(sid * ZPER, ZPER)],
            acc_hbm.at[pl.ds(cid * NROWS + sid * ZPER, ZPER)],
        )

    return passB


def _pad_w(W, b, R, ones_col):
    din, d = W.shape
    Wp = jnp.zeros((din, R), jnp.float32).at[:, :d].set(W)
    bp = jnp.zeros((R,), jnp.float32).at[:d].set(b)
    if ones_col:
        bp = bp.at[d].set(1.0)
    return Wp, bp.reshape(1, R)


def kernel(x, edge_index, batch, Wl1, bl1, Wr1, br1, att1, bias1, Wl2, bl2,
           Wr2, br2, att2, bias2, Wl3, bl3, Wr3, br3, att3, bias3, W1, b1,
           W2, b2):
    loop = jnp.arange(N, dtype=edge_index.dtype)
    fill = jnp.full((E_PAD + SLACK * CHUNK - E_TOT,), N, jnp.int32)
    src1 = jnp.concatenate([edge_index[0], loop, fill])
    dst1 = jnp.concatenate([edge_index[1], loop, fill])
    src2 = src1.reshape(NW * NCHUNK + SLACK, CHUNK)
    dst2 = dst1.reshape(NW * NCHUNK + SLACK, CHUNK)

    Rs = (128, 128, 128)
    ds = (100, 60, 30)
    layers = (
        (Wl1, bl1, Wr1, br1, att1, bias1),
        (Wl2, bl2, Wr2, br2, att2, bias2),
        (Wl3, bl3, Wr3, br3, att3, bias3),
    )

    acc = None
    for i, (Wl, bl, Wr, br, att, bias) in enumerate(layers):
        R, d = Rs[i], ds[i]
        Wlp, blp = _pad_w(Wl, bl, R, ones_col=True)
        Wrp, brp = _pad_w(Wr, br, R, ones_col=False)
        attp = jnp.zeros((R,), jnp.float32).at[:d].set(att)
        if i == 0:
            xl, xr = _tc_prep(x, Wlp, blp, Wrp, brp, R)
        else:
            bias_prev = layers[i - 1][5].reshape(1, -1)
            xl, xr = _tc_comb_prep(acc, bias_prev, Wlp, blp, Wrp, brp,
                                   ds[i - 1], R)
        KV = (d + 16) // 16
        logits, wmax = _make_passA(R, KV)(src2, dst2, xl, xr, attp)
        zeros = jnp.zeros((ZPER, R), jnp.float32)
        (acc,) = _make_passB(R, KV)(src1, dst1, logits, wmax, xl, zeros)
        acc = acc.reshape(2, NROWS, R)

    return _tc_final(acc, bias3.reshape(1, -1), batch.reshape(N, 1).astype(jnp.int32),
                     W1, b1.reshape(1, -1), W2, b2.reshape(1, -1))
